# baseline (speedup 1.0000x reference)
import jax
import jax.numpy as jnp
from jax.experimental import pallas as pl
from jax.experimental.pallas import tpu as pltpu

_DEPTH = 1000
_B0 = 16


def _onehot_body(idx_ref, out_ref, scratch, sems):
    b, s = idx_ref.shape
    nchunk = b // _B0
    iota = jax.lax.broadcasted_iota(jnp.int32, (_B0, s, _DEPTH), 2)

    def chunk(c, _):
        p = jax.lax.rem(c, 2)
        idx = idx_ref[pl.ds(c * _B0, _B0), :]
        oh = (idx[:, :, None] == iota).astype(jnp.float32)

        @pl.when(c >= 2)
        def _wait_prev():
            for k in range(_B0):
                pltpu.make_async_copy(
                    scratch.at[p, k],
                    out_ref.at[(c - 2) * _B0 + k],
                    sems.at[p, k],
                ).wait()

        scratch[p] = oh
        for k in range(_B0):
            pltpu.make_async_copy(
                scratch.at[p, k],
                out_ref.at[c * _B0 + k],
                sems.at[p, k],
            ).start()
        return 0

    jax.lax.fori_loop(0, nchunk, chunk, 0)

    def drain(i, _):
        c = nchunk - 2 + i
        p = jax.lax.rem(c, 2)
        for k in range(_B0):
            pltpu.make_async_copy(
                scratch.at[p, k],
                out_ref.at[c * _B0 + k],
                sems.at[p, k],
            ).wait()
        return 0

    jax.lax.fori_loop(0, 2, drain, 0)


def kernel(inputs):
    x = inputs.astype(jnp.int32)
    if x.ndim == 3:
        x = x[:, :, 0]
    b, s = x.shape
    return pl.pallas_call(
        _onehot_body,
        in_specs=[pl.BlockSpec(memory_space=pltpu.MemorySpace.VMEM)],
        out_specs=pl.BlockSpec(memory_space=pl.ANY),
        out_shape=jax.ShapeDtypeStruct((b, s, _DEPTH), jnp.float32),
        scratch_shapes=[
            pltpu.VMEM((2, _B0, s, _DEPTH), jnp.float32),
            pltpu.SemaphoreType.DMA((2, _B0)),
        ],
    )(x)


# speedup vs baseline: 4.4711x; 4.4711x over previous
import jax
import jax.numpy as jnp
from jax.experimental import pallas as pl

_DEPTH = 1000
_B1 = 2


def _onehot_block(idxt_ref, out_ref):
    idxt = idxt_ref[0]
    b1, b = idxt.shape
    iota = jax.lax.broadcasted_iota(jnp.int32, (b1, _DEPTH, b), 1)
    out_ref[...] = (idxt[:, None, :] == iota).astype(jnp.float32)


def kernel(inputs):
    x = inputs.astype(jnp.int32)
    if x.ndim == 3:
        x = x[:, :, 0]
    b, s = x.shape
    g = s // _B1
    xt = x.T.reshape(g, _B1, b)
    out = pl.pallas_call(
        _onehot_block,
        grid=(g,),
        in_specs=[pl.BlockSpec((1, _B1, b), lambda i: (i, 0, 0))],
        out_specs=pl.BlockSpec((_B1, _DEPTH, b), lambda i: (i, 0, 0)),
        out_shape=jax.ShapeDtypeStruct((s, _DEPTH, b), jnp.float32),
    )(xt)
    return jnp.transpose(out, (2, 0, 1))


# speedup vs baseline: 4.5137x; 1.0095x over previous
import jax
import jax.numpy as jnp
from jax.experimental import pallas as pl

_DEPTH = 1000
_B1 = 1


def _onehot_block(idxt_ref, out_ref):
    idxt = idxt_ref[0]
    b1, b = idxt.shape
    iota = jax.lax.broadcasted_iota(jnp.int32, (b1, _DEPTH, b), 1)
    out_ref[...] = (idxt[:, None, :] == iota).astype(jnp.float32)


def kernel(inputs):
    x = inputs.astype(jnp.int32)
    if x.ndim == 3:
        x = x[:, :, 0]
    b, s = x.shape
    g = s // _B1
    xt = x.T.reshape(g, _B1, b)
    out = pl.pallas_call(
        _onehot_block,
        grid=(g,),
        in_specs=[pl.BlockSpec((1, _B1, b), lambda i: (i, 0, 0))],
        out_specs=pl.BlockSpec((_B1, _DEPTH, b), lambda i: (i, 0, 0)),
        out_shape=jax.ShapeDtypeStruct((s, _DEPTH, b), jnp.float32),
    )(xt)
    return jnp.transpose(out, (2, 0, 1))
